# Initial kernel scaffold; baseline (speedup 1.0000x reference)
import jax
import jax.numpy as jnp
from jax.experimental import pallas as pl


def kernel(x, edge_index, W1, b1, W2, b2, fc1_w, fc1_b, fc2_w, fc2_b):
    raise NotImplementedError("write your pallas kernel here")


# speedup vs baseline: 13.6336x; 13.6336x over previous
import functools

import jax
import jax.numpy as jnp
from jax import lax
from jax.experimental import pallas as pl
from jax.experimental.pallas import tpu as pltpu
from jax.experimental.pallas import tpu_sc as plsc

_N = 1024
_IN = 128
_HID = 64
_E = 65536
_MAXN = 15

_NTILES = 32
_EPT = _E // _NTILES
_ROWS = 16
_COLS = _EPT // _ROWS
_ZCHUNK = _N * _N // 16

_ALPHA = 1.6732632423543772
_SCALE = 1.0507009873554805


def _selu(v):
    return _SCALE * jnp.where(v > 0, v, _ALPHA * (jnp.exp(v) - 1.0))


def _sc_adj_body(edge_hbm, zeros_hbm, out_hbm, src_v, dst_v, idx2d, val2d,
                 m_spmem):
    c = lax.axis_index("c")
    s = lax.axis_index("s")
    wid = c * 16 + s
    base = wid * _EPT

    pltpu.sync_copy(zeros_hbm, m_spmem.at[pl.ds(s * _ZCHUNK, _ZCHUNK)])
    pltpu.sync_copy(edge_hbm.at[0, pl.ds(base, _EPT)], src_v)
    pltpu.sync_copy(edge_hbm.at[1, pl.ds(base, _EPT)], dst_v)

    ones = jnp.full((16,), 1.0, jnp.float32)
    for j in range(_ROWS):
        def body(i, carry, j=j):
            t = j * _COLS + i * 16
            f = dst_v[pl.ds(t, 16)] * _N + src_v[pl.ds(t, 16)]
            idx2d[j, pl.ds(i * 16, 16)] = f
            val2d[j, pl.ds(i * 16, 16)] = ones
            return carry
        lax.fori_loop(0, _COLS // 16, body, 0)

    plsc.subcore_barrier()
    for j in range(_ROWS):
        pltpu.sync_copy(val2d.at[j], m_spmem.at[idx2d.at[j]], add=True)
    plsc.subcore_barrier()
    pltpu.sync_copy(m_spmem.at[pl.ds(s * _ZCHUNK, _ZCHUNK)],
                    out_hbm.at[c, pl.ds(s * _ZCHUNK, _ZCHUNK)])


@functools.cache
def _sc_build_adj():
    return pl.kernel(
        _sc_adj_body,
        out_type=jax.ShapeDtypeStruct((2, _N * _N), jnp.float32),
        mesh=plsc.VectorSubcoreMesh(core_axis_name="c", subcore_axis_name="s"),
        scratch_types=[
            pltpu.VMEM((_EPT,), jnp.int32),
            pltpu.VMEM((_EPT,), jnp.int32),
            pltpu.VMEM((_ROWS, _COLS), jnp.int32),
            pltpu.VMEM((_ROWS, _COLS), jnp.float32),
            pltpu.VMEM_SHARED((_N * _N,), jnp.float32),
        ],
    )


def _gcn_body(m2_ref, x_ref, w1_ref, b1_ref, w2_ref, b2_ref, h2_ref):
    m = m2_ref[0] + m2_ref[1]
    deg = jnp.sum(m, axis=1, keepdims=True) + 1.0
    dinv = lax.rsqrt(deg)
    d2 = dinv * dinv

    z = jnp.dot(x_ref[...], w1_ref[...], preferred_element_type=jnp.float32)
    y = dinv * jnp.dot(m, dinv * z, preferred_element_type=jnp.float32)
    h = _selu(y + d2 * z + b1_ref[...])

    z = jnp.dot(h, w2_ref[...], preferred_element_type=jnp.float32)
    y = dinv * jnp.dot(m, dinv * z, preferred_element_type=jnp.float32)
    h2_ref[...] = _selu(y + d2 * z + b2_ref[...])


_BK = 2048
_KSTEPS = _E // _BK


def _fc_body(h_ref, w_ref, b1_ref, w2_ref, b2_ref, o_ref, acc_ref):
    k = pl.program_id(0)

    @pl.when(k == 0)
    def _():
        acc_ref[...] = b1_ref[...]

    acc_ref[...] += jnp.dot(h_ref[...], w_ref[...],
                            preferred_element_type=jnp.float32)

    @pl.when(k == _KSTEPS - 1)
    def _():
        a = _selu(acc_ref[...])
        o_ref[...] = jnp.dot(a, w2_ref[...],
                             preferred_element_type=jnp.float32) + b2_ref[...]


def kernel(x, edge_index, W1, b1, W2, b2, fc1_w, fc1_b, fc2_w, fc2_b):
    ei = edge_index.astype(jnp.int32)
    zeros = jnp.zeros((_ZCHUNK,), jnp.float32)

    m2 = _sc_build_adj()(ei, zeros).reshape(2, _N, _N)

    h2 = pl.pallas_call(
        _gcn_body,
        out_shape=jax.ShapeDtypeStruct((_N, _HID), jnp.float32),
    )(m2, x, W1, b1.reshape(1, _HID), W2, b2.reshape(1, _HID))

    hflat = h2.reshape(1, _E)
    out = pl.pallas_call(
        _fc_body,
        grid=(_KSTEPS,),
        in_specs=[
            pl.BlockSpec((1, _BK), lambda k: (0, k)),
            pl.BlockSpec((_BK, _N), lambda k: (k, 0)),
            pl.BlockSpec((1, _N), lambda k: (0, 0)),
            pl.BlockSpec((_N, _MAXN), lambda k: (0, 0)),
            pl.BlockSpec((1, _MAXN), lambda k: (0, 0)),
        ],
        out_specs=pl.BlockSpec((1, _MAXN), lambda k: (0, 0)),
        out_shape=jax.ShapeDtypeStruct((1, _MAXN), jnp.float32),
        scratch_shapes=[pltpu.VMEM((1, _N), jnp.float32)],
    )(hflat, fc1_w, fc1_b.reshape(1, _N), fc2_w, fc2_b.reshape(1, _MAXN))
    return out


# speedup vs baseline: 14.6468x; 1.0743x over previous
import functools

import jax
import jax.numpy as jnp
from jax import lax
from jax.experimental import pallas as pl
from jax.experimental.pallas import tpu as pltpu
from jax.experimental.pallas import tpu_sc as plsc

_N = 1024
_IN = 128
_HID = 64
_E = 65536
_MAXN = 15

_NTILES = 32
_EPT = _E // _NTILES
_ROWS = 16
_COLS = _EPT // _ROWS
_ZCHUNK = _N * _N // 16

_ALPHA = 1.6732632423543772
_SCALE = 1.0507009873554805


def _selu(v):
    return _SCALE * jnp.where(v > 0, v, _ALPHA * (jnp.exp(v) - 1.0))


def _sc_adj_body(edge_hbm, zeros_hbm, out_hbm, src_v, dst_v, idx2d, val2d,
                 m_spmem):
    c = lax.axis_index("c")
    s = lax.axis_index("s")
    wid = c * 16 + s
    base = wid * _EPT

    pltpu.sync_copy(zeros_hbm, m_spmem.at[pl.ds(s * _ZCHUNK, _ZCHUNK)])
    pltpu.sync_copy(edge_hbm.at[0, pl.ds(base, _EPT)], src_v)
    pltpu.sync_copy(edge_hbm.at[1, pl.ds(base, _EPT)], dst_v)

    ones = jnp.full((16,), 1.0, jnp.float32)
    for j in range(_ROWS):
        def body(i, carry, j=j):
            t = j * _COLS + i * 16
            src = src_v[pl.ds(t, 16)]
            dst = dst_v[pl.ds(t, 16)]
            f = ((src & ~127) * _N + dst * 128 + (src & 127))
            idx2d[j, pl.ds(i * 16, 16)] = f
            val2d[j, pl.ds(i * 16, 16)] = ones
            return carry
        lax.fori_loop(0, _COLS // 16, body, 0)

    plsc.subcore_barrier()
    for j in range(_ROWS):
        pltpu.sync_copy(val2d.at[j], m_spmem.at[idx2d.at[j]], add=True)
    plsc.subcore_barrier()
    pltpu.sync_copy(m_spmem.at[pl.ds(s * _ZCHUNK, _ZCHUNK)],
                    out_hbm.at[pl.ds((c * 16 + s) * _ZCHUNK, _ZCHUNK)])


@functools.cache
def _sc_build_adj():
    return pl.kernel(
        _sc_adj_body,
        out_type=jax.ShapeDtypeStruct((2 * _N * _N,), jnp.float32),
        mesh=plsc.VectorSubcoreMesh(core_axis_name="c", subcore_axis_name="s"),
        scratch_types=[
            pltpu.VMEM((_EPT,), jnp.int32),
            pltpu.VMEM((_EPT,), jnp.int32),
            pltpu.VMEM((_ROWS, _COLS), jnp.int32),
            pltpu.VMEM((_ROWS, _COLS), jnp.float32),
            pltpu.VMEM_SHARED((_N * _N,), jnp.float32),
        ],
    )


def _gcn_body(m2_ref, x_ref, w1_ref, b1_ref, w2_ref, b2_ref, h2_ref, ms_ref):
    for k in range(8):
        ms_ref[k] = m2_ref[0, k] + m2_ref[1, k]

    deg = jnp.zeros((_N, 1), jnp.float32) + 1.0
    for k in range(8):
        deg = deg + jnp.sum(ms_ref[k], axis=1, keepdims=True)
    dinv = lax.rsqrt(deg)
    d2 = dinv * dinv

    def agg(u):
        y = jnp.dot(ms_ref[0], u[0:128],
                    preferred_element_type=jnp.float32)
        for k in range(1, 8):
            y = y + jnp.dot(ms_ref[k], u[128 * k:128 * (k + 1)],
                            preferred_element_type=jnp.float32)
        return y

    z = jnp.dot(x_ref[...], w1_ref[...], preferred_element_type=jnp.float32)
    h = _selu(dinv * agg(dinv * z) + d2 * z + b1_ref[...])

    z = jnp.dot(h, w2_ref[...], preferred_element_type=jnp.float32)
    h2_ref[...] = _selu(dinv * agg(dinv * z) + d2 * z + b2_ref[...])


_BK = 4096
_KSTEPS = _E // _BK


def _fc_body(h_ref, w_ref, b1_ref, w2_ref, b2_ref, o_ref, acc_ref):
    k = pl.program_id(0)

    @pl.when(k == 0)
    def _():
        acc_ref[...] = b1_ref[...]

    acc_ref[...] += jnp.dot(h_ref[...], w_ref[...],
                            preferred_element_type=jnp.float32)

    @pl.when(k == _KSTEPS - 1)
    def _():
        a = _selu(acc_ref[...])
        o_ref[...] = jnp.dot(a, w2_ref[...],
                             preferred_element_type=jnp.float32) + b2_ref[...]


def kernel(x, edge_index, W1, b1, W2, b2, fc1_w, fc1_b, fc2_w, fc2_b):
    ei = edge_index.astype(jnp.int32)
    zeros = jnp.zeros((_ZCHUNK,), jnp.float32)

    m2 = _sc_build_adj()(ei, zeros).reshape(2, 8, _N, 128)

    h2 = pl.pallas_call(
        _gcn_body,
        out_shape=jax.ShapeDtypeStruct((_N, _HID), jnp.float32),
        scratch_shapes=[pltpu.VMEM((8, _N, 128), jnp.float32)],
    )(m2, x, W1, b1.reshape(1, _HID), W2, b2.reshape(1, _HID))

    hflat = h2.reshape(1, _E)
    out = pl.pallas_call(
        _fc_body,
        grid=(_KSTEPS,),
        in_specs=[
            pl.BlockSpec((1, _BK), lambda k: (0, k)),
            pl.BlockSpec((_BK, _N), lambda k: (k, 0)),
            pl.BlockSpec((1, _N), lambda k: (0, 0)),
            pl.BlockSpec((_N, _MAXN), lambda k: (0, 0)),
            pl.BlockSpec((1, _MAXN), lambda k: (0, 0)),
        ],
        out_specs=pl.BlockSpec((1, _MAXN), lambda k: (0, 0)),
        out_shape=jax.ShapeDtypeStruct((1, _MAXN), jnp.float32),
        scratch_shapes=[pltpu.VMEM((1, _N), jnp.float32)],
    )(hflat, fc1_w, fc1_b.reshape(1, _N), fc2_w, fc2_b.reshape(1, _MAXN))
    return out
